# Initial kernel scaffold; baseline (speedup 1.0000x reference)
import jax
import jax.numpy as jnp
from jax import lax
from jax.experimental import pallas as pl
from jax.experimental.pallas import tpu as pltpu
from jax.experimental.pallas import tpu_sc as plsc

N_P = 50000
N_GO = 10000
E = 500000
D = 128

NG = 4
CW = D // NG

NC = 2
NS = 16
LANES = 128
CHUNK_ROWS = 8
CHUNK = CHUNK_ROWS * LANES
CHUNKS_PER_SUB = 31
E_PAD = NS * CHUNKS_PER_SUB * CHUNK
IDX_ROWS = E_PAD // LANES
ROWS_PER_SUB = N_P // NS
ACC_ROWS = N_P + 16
DUMP_ROW = N_P + 8
ZB_ROWS = 128
ZERO_COPY_ROWS = 125

MB = 2000


def _matmul_kernel(h_ref, w_ref, o0, o1, o2, o3):
    f = lax.dot_general(h_ref[...], w_ref[...], (((1,), (0,)), ((), ())),
                        precision=lax.Precision.HIGHEST,
                        preferred_element_type=jnp.float32)
    outs = (o0, o1, o2, o3)
    for g in range(NG):
        outs[g][...] = f[:, g * CW:(g + 1) * CW]


def _feat_tables(h, W):
    grid = (N_GO // MB,)
    out_shapes = tuple(jax.ShapeDtypeStruct((N_GO, CW), jnp.float32)
                       for _ in range(NG))
    return pl.pallas_call(
        _matmul_kernel,
        grid=grid,
        in_specs=[
            pl.BlockSpec((MB, D), lambda i: (i, 0)),
            pl.BlockSpec((D, D), lambda i: (0, 0)),
        ],
        out_specs=tuple(pl.BlockSpec((MB, CW), lambda i: (i, 0))
                        for _ in range(NG)),
        out_shape=out_shapes,
    )(h, W)


def _sc_kernel(*refs):
    tables = refs[0:12]
    srcs = refs[12:15]
    dsts = refs[15:18]
    tmp = refs[18]
    src_v, dst_v, rows_v, zb, acc, sem = refs[19:25]

    core = lax.axis_index("c")
    sub = lax.axis_index("s")

    zrow = jnp.zeros((16,), jnp.float32)
    for r in range(ZB_ROWS):
        zb[r, 0:16] = zrow
        zb[r, 16:32] = zrow

    def round_body(tab, src_hbm, dst_hbm, out_base):
        @pl.loop(0, ROWS_PER_SUB // ZERO_COPY_ROWS)
        def _(i):
            pltpu.sync_copy(
                zb.at[pl.ds(0, ZERO_COPY_ROWS)],
                acc.at[pl.ds(sub * ROWS_PER_SUB + i * ZERO_COPY_ROWS,
                             ZERO_COPY_ROWS)])
        plsc.subcore_barrier()

        @pl.loop(0, CHUNKS_PER_SUB)
        def _(j):
            row0 = sub * (CHUNKS_PER_SUB * CHUNK_ROWS) + j * CHUNK_ROWS
            pltpu.sync_copy(src_hbm.at[pl.ds(row0, CHUNK_ROWS)], src_v)
            pltpu.sync_copy(dst_hbm.at[pl.ds(row0, CHUNK_ROWS)], dst_v)
            cps = [pltpu.async_copy(tab.at[src_v.at[k]], rows_v.at[k], sem)
                   for k in range(CHUNK_ROWS)]
            for cp in cps:
                cp.wait()
            for k in range(CHUNK_ROWS):
                pltpu.sync_copy(rows_v.at[k], acc.at[dst_v.at[k]], add=True)
        plsc.subcore_barrier()

        pltpu.sync_copy(
            acc.at[pl.ds(sub * ROWS_PER_SUB, ROWS_PER_SUB)],
            tmp.at[pl.ds(out_base + sub * ROWS_PER_SUB, ROWS_PER_SUB)])

    for rel in range(3):
        for g in range(NG):
            @pl.when(core == (g // 2))
            def _(rel=rel, g=g):
                round_body(tables[rel * NG + g], srcs[rel], dsts[rel],
                           (rel * NG + g) * N_P)


def _sc_aggregate(tables, srcs, dsts):
    mesh = plsc.VectorSubcoreMesh(core_axis_name="c", subcore_axis_name="s")
    kern = pl.kernel(
        _sc_kernel,
        out_type=jax.ShapeDtypeStruct((3 * NG * N_P, CW), jnp.float32),
        mesh=mesh,
        scratch_types=[
            pltpu.VMEM((CHUNK_ROWS, LANES), jnp.int32),
            pltpu.VMEM((CHUNK_ROWS, LANES), jnp.int32),
            pltpu.VMEM((CHUNK_ROWS, LANES, CW), jnp.float32),
            pltpu.VMEM((ZB_ROWS, CW), jnp.float32),
            pltpu.VMEM_SHARED((ACC_ROWS, CW), jnp.float32),
            pltpu.SemaphoreType.DMA,
        ],
    )
    return kern(*tables, *srcs, *dsts)


def _combine_kernel(mf_ref, bp_ref, cc_ref, b_ref, o_ref):
    for g in range(NG):
        sl = slice(g * CW, (g + 1) * CW)
        o_ref[:, sl] = (
            jnp.maximum(mf_ref[g] + b_ref[0, sl], 0.0)
            + jnp.maximum(bp_ref[g] + b_ref[1, sl], 0.0)
            + jnp.maximum(cc_ref[g] + b_ref[2, sl], 0.0))


def _combine(tmp, b3):
    tmp = tmp.reshape(3 * NG, N_P, CW)
    grid = (N_P // MB,)
    in_specs = [
        pl.BlockSpec((NG, MB, CW), lambda i: (0, i, 0)),
        pl.BlockSpec((NG, MB, CW), lambda i: (1, i, 0)),
        pl.BlockSpec((NG, MB, CW), lambda i: (2, i, 0)),
        pl.BlockSpec((3, D), lambda i: (0, 0)),
    ]
    return pl.pallas_call(
        _combine_kernel,
        grid=grid,
        in_specs=in_specs,
        out_specs=pl.BlockSpec((MB, D), lambda i: (i, 0)),
        out_shape=jax.ShapeDtypeStruct((N_P, D), jnp.float32),
    )(tmp, tmp, tmp, b3)


def _pad_idx(idx, fill):
    idx = idx.astype(jnp.int32)
    pad = jnp.full((E_PAD - E,), fill, jnp.int32)
    return jnp.concatenate([idx, pad]).reshape(IDX_ROWS, LANES)


@jax.jit
def _impl(h_mf, h_bp, h_cc, src_mf, dst_mf, src_bp, dst_bp, src_cc, dst_cc,
          W_mf, b_mf, W_bp, b_bp, W_cc, b_cc):
    tables = []
    for h, W in ((h_mf, W_mf), (h_bp, W_bp), (h_cc, W_cc)):
        tables.extend(_feat_tables(h, W))
    srcs = [_pad_idx(s, 0) for s in (src_mf, src_bp, src_cc)]
    dsts = [_pad_idx(d, DUMP_ROW) for d in (dst_mf, dst_bp, dst_cc)]
    tmp = _sc_aggregate(tables, srcs, dsts)
    b3 = jnp.stack([b_mf, b_bp, b_cc])
    return _combine(tmp, b3)


def kernel(h_p, h_mf, h_bp, h_cc, src_mf, dst_mf, src_bp, dst_bp,
           src_cc, dst_cc, W_mf, b_mf, W_bp, b_bp, W_cc, b_cc):
    del h_p
    return _impl(h_mf, h_bp, h_cc, src_mf, dst_mf, src_bp, dst_bp,
                 src_cc, dst_cc, W_mf, b_mf, W_bp, b_bp, W_cc, b_cc)


# speedup vs baseline: 2.3199x; 2.3199x over previous
import jax
import jax.numpy as jnp
from jax import lax
from jax.experimental import pallas as pl
from jax.experimental.pallas import tpu as pltpu
from jax.experimental.pallas import tpu_sc as plsc

N_P = 50000
N_GO = 10000
E = 500000
D = 128

NG = 8
CW = D // NG

NC = 2
NS = 16
LANES = 128
CHUNK_ROWS = 8
CHUNK = CHUNK_ROWS * LANES
CHUNKS_PER_SUB = 31
E_PAD = NS * CHUNKS_PER_SUB * CHUNK
IDX_ROWS = E_PAD // LANES
ROWS_PER_SUB = 3128
NP_PAD = NS * ROWS_PER_SUB
ACC_ROWS = NP_PAD + 16
DUMP_ROW = NP_PAD + 8
ZB_ROWS = 136
N_ZERO_COPIES = ROWS_PER_SUB // ZB_ROWS

MB = 2000


def _matmul_kernel(h_ref, w_ref, *outs):
    f = lax.dot_general(h_ref[...], w_ref[...], (((1,), (0,)), ((), ())),
                        precision=lax.Precision.HIGHEST,
                        preferred_element_type=jnp.float32)
    for g in range(NG):
        outs[g][...] = f[:, g * CW:(g + 1) * CW]


def _feat_tables(h, W):
    grid = (N_GO // MB,)
    out_shapes = tuple(jax.ShapeDtypeStruct((N_GO, CW), jnp.float32)
                       for _ in range(NG))
    return pl.pallas_call(
        _matmul_kernel,
        grid=grid,
        in_specs=[
            pl.BlockSpec((MB, D), lambda i: (i, 0)),
            pl.BlockSpec((D, D), lambda i: (0, 0)),
        ],
        out_specs=tuple(pl.BlockSpec((MB, CW), lambda i: (i, 0))
                        for _ in range(NG)),
        out_shape=out_shapes,
    )(h, W)


def _sc_kernel(*refs):
    nt = 3 * NG
    tables = refs[0:nt]
    srcs = refs[nt:nt + 3]
    dsts = refs[nt + 3:nt + 6]
    tmp = refs[nt + 6]
    src_v, dst_v, rows_v, zb, acc, sem = refs[nt + 7:nt + 13]

    core = lax.axis_index("c")
    sub = lax.axis_index("s")

    zrow = jnp.zeros((16,), jnp.float32)
    @pl.loop(0, ZB_ROWS)
    def _(r):
        zb[r, 0:CW] = zrow

    def round_body(tab, src_hbm, dst_hbm, out_base):
        @pl.loop(0, N_ZERO_COPIES)
        def _(i):
            pltpu.sync_copy(
                zb.at[pl.ds(0, ZB_ROWS)],
                acc.at[pl.ds(sub * ROWS_PER_SUB + i * ZB_ROWS, ZB_ROWS)])
        @pl.when(sub == 0)
        def _():
            pltpu.sync_copy(zb.at[pl.ds(0, ACC_ROWS - NP_PAD)],
                            acc.at[pl.ds(NP_PAD, ACC_ROWS - NP_PAD)])
        plsc.subcore_barrier()

        @pl.loop(0, CHUNKS_PER_SUB)
        def _(j):
            row0 = sub * (CHUNKS_PER_SUB * CHUNK_ROWS) + j * CHUNK_ROWS
            pltpu.sync_copy(src_hbm.at[pl.ds(row0, CHUNK_ROWS)], src_v)
            pltpu.sync_copy(dst_hbm.at[pl.ds(row0, CHUNK_ROWS)], dst_v)
            cps = [pltpu.async_copy(tab.at[src_v.at[k]],
                                    rows_v.at[pl.ds(k * LANES, LANES)], sem)
                   for k in range(CHUNK_ROWS)]
            for cp in cps:
                cp.wait()
            for k in range(CHUNK_ROWS):
                pltpu.sync_copy(rows_v.at[pl.ds(k * LANES, LANES)],
                                acc.at[dst_v.at[k]], add=True)
        plsc.subcore_barrier()

        pltpu.sync_copy(
            acc.at[pl.ds(sub * ROWS_PER_SUB, ROWS_PER_SUB)],
            tmp.at[pl.ds(out_base + sub * ROWS_PER_SUB, ROWS_PER_SUB)])

    for rel in range(3):
        for g in range(NG):
            @pl.when(core == (g // (NG // NC)))
            def _(rel=rel, g=g):
                round_body(tables[rel * NG + g], srcs[rel], dsts[rel],
                           (rel * NG + g) * NP_PAD)


def _sc_aggregate(tables, srcs, dsts):
    mesh = plsc.VectorSubcoreMesh(core_axis_name="c", subcore_axis_name="s")
    kern = pl.kernel(
        _sc_kernel,
        out_type=jax.ShapeDtypeStruct((3 * NG * NP_PAD, CW), jnp.float32),
        mesh=mesh,
        compiler_params=pltpu.CompilerParams(use_tc_tiling_on_sc=False),
        scratch_types=[
            pltpu.VMEM((CHUNK_ROWS, LANES), jnp.int32),
            pltpu.VMEM((CHUNK_ROWS, LANES), jnp.int32),
            pltpu.VMEM((CHUNK, CW), jnp.float32),
            pltpu.VMEM((ZB_ROWS, CW), jnp.float32),
            pltpu.VMEM_SHARED((ACC_ROWS, CW), jnp.float32),
            pltpu.SemaphoreType.DMA,
        ],
    )
    return kern(*tables, *srcs, *dsts)


def _combine_kernel(mf_ref, bp_ref, cc_ref, b_ref, o_ref):
    for g in range(NG):
        sl = slice(g * CW, (g + 1) * CW)
        o_ref[:, sl] = (
            jnp.maximum(mf_ref[g] + b_ref[0, sl], 0.0)
            + jnp.maximum(bp_ref[g] + b_ref[1, sl], 0.0)
            + jnp.maximum(cc_ref[g] + b_ref[2, sl], 0.0))


def _combine(tmp, b3):
    tmp = tmp.reshape(3 * NG, NP_PAD, CW)
    grid = (N_P // MB,)
    in_specs = [
        pl.BlockSpec((NG, MB, CW), lambda i: (0, i, 0)),
        pl.BlockSpec((NG, MB, CW), lambda i: (1, i, 0)),
        pl.BlockSpec((NG, MB, CW), lambda i: (2, i, 0)),
        pl.BlockSpec((3, D), lambda i: (0, 0)),
    ]
    return pl.pallas_call(
        _combine_kernel,
        grid=grid,
        in_specs=in_specs,
        out_specs=pl.BlockSpec((MB, D), lambda i: (i, 0)),
        out_shape=jax.ShapeDtypeStruct((N_P, D), jnp.float32),
    )(tmp, tmp, tmp, b3)


def _pad_idx(idx, fill):
    idx = idx.astype(jnp.int32)
    pad = jnp.full((E_PAD - E,), fill, jnp.int32)
    return jnp.concatenate([idx, pad]).reshape(IDX_ROWS, LANES)


@jax.jit
def _impl(h_mf, h_bp, h_cc, src_mf, dst_mf, src_bp, dst_bp, src_cc, dst_cc,
          W_mf, b_mf, W_bp, b_bp, W_cc, b_cc):
    tables = []
    for h, W in ((h_mf, W_mf), (h_bp, W_bp), (h_cc, W_cc)):
        tables.extend(_feat_tables(h, W))
    srcs = [_pad_idx(s, 0) for s in (src_mf, src_bp, src_cc)]
    dsts = [_pad_idx(d, DUMP_ROW) for d in (dst_mf, dst_bp, dst_cc)]
    tmp = _sc_aggregate(tables, srcs, dsts)
    b3 = jnp.stack([b_mf, b_bp, b_cc])
    return _combine(tmp, b3)


def kernel(h_p, h_mf, h_bp, h_cc, src_mf, dst_mf, src_bp, dst_bp,
           src_cc, dst_cc, W_mf, b_mf, W_bp, b_bp, W_cc, b_cc):
    del h_p
    return _impl(h_mf, h_bp, h_cc, src_mf, dst_mf, src_bp, dst_bp,
                 src_cc, dst_cc, W_mf, b_mf, W_bp, b_bp, W_cc, b_cc)


# speedup vs baseline: 3.9387x; 1.6978x over previous
import jax
import jax.numpy as jnp
from jax import lax
from jax.experimental import pallas as pl
from jax.experimental.pallas import tpu as pltpu
from jax.experimental.pallas import tpu_sc as plsc

N_P = 50000
N_GO = 10000
E = 500000
D = 128

NG = 8
CW = D // NG
NC = 2
NS = 16
LANES = 128
CHUNK_ROWS = 8
CHUNK = CHUNK_ROWS * LANES
CHUNKS_PER_SUB = 31
E_PAD = NS * CHUNKS_PER_SUB * CHUNK
IDX_ROWS = E_PAD // LANES
ROWS_PER_SUB = 3128
NP_PAD = NS * ROWS_PER_SUB
ACC_ROWS = NP_PAD + 16
ZB_ROWS = 136
N_ZERO_COPIES = ROWS_PER_SUB // ZB_ROWS
TAB_SPAN = 632
TAB_TAIL = N_GO - 15 * TAB_SPAN

MB = 2000


def _matmul_kernel(h_ref, w_ref, *outs):
    f = lax.dot_general(h_ref[...], w_ref[...], (((1,), (0,)), ((), ())),
                        precision=lax.Precision.HIGHEST,
                        preferred_element_type=jnp.float32)
    for g in range(NG):
        outs[g][...] = f[:, g * CW:(g + 1) * CW]


def _feat_tables(h, W):
    grid = (N_GO // MB,)
    out_shapes = tuple(jax.ShapeDtypeStruct((N_GO, CW), jnp.float32)
                       for _ in range(NG))
    return pl.pallas_call(
        _matmul_kernel,
        grid=grid,
        in_specs=[
            pl.BlockSpec((MB, D), lambda i: (i, 0)),
            pl.BlockSpec((D, D), lambda i: (0, 0)),
        ],
        out_specs=tuple(pl.BlockSpec((MB, CW), lambda i: (i, 0))
                        for _ in range(NG)),
        out_shape=out_shapes,
    )(h, W)


def _sc_kernel(*refs):
    nt = 3 * NG
    tables = refs[0:nt]
    srcs = refs[nt:nt + 3]
    dsts = refs[nt + 3:nt + 6]
    tmp = refs[nt + 6]
    src_v, dst_v, rows_v, zb, tab_s, acc, sem = refs[nt + 7:nt + 14]

    core = lax.axis_index("c")
    sub = lax.axis_index("s")

    zrow = jnp.zeros((16,), jnp.float32)
    @pl.loop(0, ZB_ROWS)
    def _(r):
        zb[r, 0:CW] = zrow

    def round_body(tab, src_hbm, dst_hbm, rel, g):
        @pl.when(sub < 15)
        def _():
            pltpu.sync_copy(tab.at[pl.ds(sub * TAB_SPAN, TAB_SPAN)],
                            tab_s.at[pl.ds(sub * TAB_SPAN, TAB_SPAN)])
        @pl.when(sub == 15)
        def _():
            pltpu.sync_copy(tab.at[pl.ds(15 * TAB_SPAN, TAB_TAIL)],
                            tab_s.at[pl.ds(15 * TAB_SPAN, TAB_TAIL)])
        @pl.loop(0, N_ZERO_COPIES)
        def _(i):
            pltpu.sync_copy(
                zb.at[pl.ds(0, ZB_ROWS)],
                acc.at[pl.ds(sub * ROWS_PER_SUB + i * ZB_ROWS, ZB_ROWS)])
        @pl.when(sub == 0)
        def _():
            pltpu.sync_copy(zb.at[pl.ds(0, ACC_ROWS - NP_PAD)],
                            acc.at[pl.ds(NP_PAD, ACC_ROWS - NP_PAD)])
        plsc.subcore_barrier()

        @pl.loop(0, CHUNKS_PER_SUB)
        def _(j):
            row0 = sub * (CHUNKS_PER_SUB * CHUNK_ROWS) + j * CHUNK_ROWS
            pltpu.sync_copy(src_hbm.at[pl.ds(row0, CHUNK_ROWS)], src_v)
            pltpu.sync_copy(dst_hbm.at[pl.ds(row0, CHUNK_ROWS)], dst_v)
            cps = [pltpu.async_copy(tab_s.at[src_v.at[k]],
                                    rows_v.at[pl.ds(k * LANES, LANES)], sem)
                   for k in range(CHUNK_ROWS)]
            for cp in cps:
                cp.wait()
            for k in range(CHUNK_ROWS):
                pltpu.sync_copy(rows_v.at[pl.ds(k * LANES, LANES)],
                                acc.at[dst_v.at[k]], add=True)
        plsc.subcore_barrier()

        pltpu.sync_copy(
            acc.at[pl.ds(sub * ROWS_PER_SUB, ROWS_PER_SUB)],
            tmp.at[pl.ds(rel * NP_PAD + sub * ROWS_PER_SUB, ROWS_PER_SUB),
                   pl.ds(g * CW, CW)])

    for rel in range(3):
        for g in range(NG):
            @pl.when(core == (g // (NG // NC)))
            def _(rel=rel, g=g):
                round_body(tables[rel * NG + g], srcs[rel], dsts[rel],
                           rel, g)


def _sc_aggregate(tables, srcs, dsts):
    mesh = plsc.VectorSubcoreMesh(core_axis_name="c", subcore_axis_name="s")
    kern = pl.kernel(
        _sc_kernel,
        out_type=jax.ShapeDtypeStruct((3 * NP_PAD, D), jnp.float32),
        mesh=mesh,
        compiler_params=pltpu.CompilerParams(use_tc_tiling_on_sc=False),
        scratch_types=[
            pltpu.VMEM((CHUNK_ROWS, LANES), jnp.int32),
            pltpu.VMEM((CHUNK_ROWS, LANES), jnp.int32),
            pltpu.VMEM((CHUNK, CW), jnp.float32),
            pltpu.VMEM((ZB_ROWS, CW), jnp.float32),
            pltpu.VMEM_SHARED((N_GO, CW), jnp.float32),
            pltpu.VMEM_SHARED((ACC_ROWS, CW), jnp.float32),
            pltpu.SemaphoreType.DMA,
        ],
    )
    return kern(*tables, *srcs, *dsts)


def _combine_kernel(mf_ref, bp_ref, cc_ref, b_ref, o_ref):
    o_ref[...] = (
        jnp.maximum(mf_ref[0] + b_ref[0], 0.0)
        + jnp.maximum(bp_ref[0] + b_ref[1], 0.0)
        + jnp.maximum(cc_ref[0] + b_ref[2], 0.0))


def _combine(tmp, b3):
    tmp = tmp.reshape(3, NP_PAD, D)
    grid = (N_P // MB,)
    in_specs = [
        pl.BlockSpec((1, MB, D), lambda i: (0, i, 0)),
        pl.BlockSpec((1, MB, D), lambda i: (1, i, 0)),
        pl.BlockSpec((1, MB, D), lambda i: (2, i, 0)),
        pl.BlockSpec((3, D), lambda i: (0, 0)),
    ]
    return pl.pallas_call(
        _combine_kernel,
        grid=grid,
        in_specs=in_specs,
        out_specs=pl.BlockSpec((MB, D), lambda i: (i, 0)),
        out_shape=jax.ShapeDtypeStruct((N_P, D), jnp.float32),
    )(tmp, tmp, tmp, b3)


def _pad_idx(idx, fill):
    idx = idx.astype(jnp.int32)
    return jnp.concatenate([idx, fill]).reshape(IDX_ROWS, LANES)


@jax.jit
def _impl(h_mf, h_bp, h_cc, src_mf, dst_mf, src_bp, dst_bp, src_cc, dst_cc,
          W_mf, b_mf, W_bp, b_bp, W_cc, b_cc):
    tables = []
    for h, W in ((h_mf, W_mf), (h_bp, W_bp), (h_cc, W_cc)):
        tables.extend(_feat_tables(h, W))
    ar = jnp.arange(E_PAD - E, dtype=jnp.int32)
    src_fill = ar % N_GO
    dst_fill = NP_PAD + (ar % (ACC_ROWS - NP_PAD))
    srcs = [_pad_idx(s, src_fill) for s in (src_mf, src_bp, src_cc)]
    dsts = [_pad_idx(d, dst_fill) for d in (dst_mf, dst_bp, dst_cc)]
    tmp = _sc_aggregate(tables, srcs, dsts)
    b3 = jnp.stack([b_mf, b_bp, b_cc])
    return _combine(tmp, b3)


def kernel(h_p, h_mf, h_bp, h_cc, src_mf, dst_mf, src_bp, dst_bp,
           src_cc, dst_cc, W_mf, b_mf, W_bp, b_bp, W_cc, b_cc):
    del h_p
    return _impl(h_mf, h_bp, h_cc, src_mf, dst_mf, src_bp, dst_bp,
                 src_cc, dst_cc, W_mf, b_mf, W_bp, b_bp, W_cc, b_cc)


# speedup vs baseline: 5.0747x; 1.2884x over previous
import jax
import jax.numpy as jnp
from jax import lax
from jax.experimental import pallas as pl
from jax.experimental.pallas import tpu as pltpu
from jax.experimental.pallas import tpu_sc as plsc

N_P = 50000
N_GO = 10000
E = 500000
D = 128

NG = 8
CW = D // NG
NC = 2
NS = 16
LANES = 128
CHUNK_ROWS = 8
CHUNK = CHUNK_ROWS * LANES
CHUNKS_PER_SUB = 31
E_PAD = NS * CHUNKS_PER_SUB * CHUNK
IDX_ROWS = E_PAD // LANES
ROWS_PER_SUB = 3128
NP_PAD = NS * ROWS_PER_SUB
ACC_ROWS = NP_PAD + 16
ZB_ROWS = 136
N_ZERO_COPIES = ROWS_PER_SUB // ZB_ROWS
TAB_SPAN = 632
TAB_TAIL = N_GO - 15 * TAB_SPAN

MB = 2000


def _matmul_kernel(h_ref, w_ref, *outs):
    f = lax.dot_general(h_ref[...], w_ref[...], (((1,), (0,)), ((), ())),
                        precision=lax.Precision.HIGHEST,
                        preferred_element_type=jnp.float32)
    for g in range(NG):
        outs[g][...] = f[:, g * CW:(g + 1) * CW]


def _feat_tables(h, W):
    grid = (N_GO // MB,)
    out_shapes = tuple(jax.ShapeDtypeStruct((N_GO, CW), jnp.float32)
                       for _ in range(NG))
    return pl.pallas_call(
        _matmul_kernel,
        grid=grid,
        in_specs=[
            pl.BlockSpec((MB, D), lambda i: (i, 0)),
            pl.BlockSpec((D, D), lambda i: (0, 0)),
        ],
        out_specs=tuple(pl.BlockSpec((MB, CW), lambda i: (i, 0))
                        for _ in range(NG)),
        out_shape=out_shapes,
    )(h, W)


def _sc_kernel(*refs):
    nt = 3 * NG
    tables = refs[0:nt]
    srcs = refs[nt:nt + 3]
    dsts = refs[nt + 3:nt + 6]
    tmp = refs[nt + 6]
    src_v, dst_v, rows_v, zb, tab_s, acc, sem, isem = refs[nt + 7:nt + 15]

    core = lax.axis_index("c")
    sub = lax.axis_index("s")

    zrow = jnp.zeros((16,), jnp.float32)
    @pl.loop(0, ZB_ROWS)
    def _(r):
        zb[r, 0:CW] = zrow

    def round_body(tab, src_hbm, dst_hbm, rel, g):
        @pl.when(sub < 15)
        def _():
            pltpu.sync_copy(tab.at[pl.ds(sub * TAB_SPAN, TAB_SPAN)],
                            tab_s.at[pl.ds(sub * TAB_SPAN, TAB_SPAN)])
        @pl.when(sub == 15)
        def _():
            pltpu.sync_copy(tab.at[pl.ds(15 * TAB_SPAN, TAB_TAIL)],
                            tab_s.at[pl.ds(15 * TAB_SPAN, TAB_TAIL)])
        @pl.loop(0, N_ZERO_COPIES)
        def _(i):
            pltpu.sync_copy(
                zb.at[pl.ds(0, ZB_ROWS)],
                acc.at[pl.ds(sub * ROWS_PER_SUB + i * ZB_ROWS, ZB_ROWS)])
        @pl.when(sub == 0)
        def _():
            pltpu.sync_copy(zb.at[pl.ds(0, ACC_ROWS - NP_PAD)],
                            acc.at[pl.ds(NP_PAD, ACC_ROWS - NP_PAD)])
        plsc.subcore_barrier()

        base = sub * (CHUNKS_PER_SUB * CHUNK_ROWS)

        def fire_idx(chunk, buf, how):
            row0 = base + chunk * CHUNK_ROWS
            how(src_hbm.at[pl.ds(row0, CHUNK_ROWS)],
                src_v.at[pl.ds(buf * CHUNK_ROWS, CHUNK_ROWS)])
            how(dst_hbm.at[pl.ds(row0, CHUNK_ROWS)],
                dst_v.at[pl.ds(buf * CHUNK_ROWS, CHUNK_ROWS)])

        fire_idx(0, 0, pltpu.sync_copy)
        fire_idx(1, 1, pltpu.sync_copy)
        for k in range(CHUNK_ROWS):
            pltpu.async_copy(tab_s.at[src_v.at[k]],
                             rows_v.at[pl.ds(k * LANES, LANES)], sem)

        @pl.loop(0, CHUNKS_PER_SUB)
        def _(j):
            b0 = lax.rem(j, 3)
            b1 = lax.rem(j + 1, 3)
            b2 = lax.rem(j + 2, 3)
            p0 = lax.rem(j, 2) * CHUNK
            p1 = lax.rem(j + 1, 2) * CHUNK
            @pl.when(jnp.logical_and(j >= 1, j + 1 < CHUNKS_PER_SUB))
            def _():
                for _i in range(2):
                    pltpu.make_async_copy(
                        src_hbm.at[pl.ds(base, CHUNK_ROWS)],
                        src_v.at[pl.ds(0, CHUNK_ROWS)], isem).wait()
            @pl.when(j + 2 < CHUNKS_PER_SUB)
            def _():
                fire_idx(j + 2, b2,
                         lambda s, d: pltpu.async_copy(s, d, isem))
            for k in range(CHUNK_ROWS):
                pltpu.make_async_copy(
                    tab_s.at[src_v.at[b0 * CHUNK_ROWS + k]],
                    rows_v.at[pl.ds(p0 + k * LANES, LANES)], sem).wait()
            @pl.when(j + 1 < CHUNKS_PER_SUB)
            def _():
                for k in range(CHUNK_ROWS):
                    pltpu.async_copy(
                        tab_s.at[src_v.at[b1 * CHUNK_ROWS + k]],
                        rows_v.at[pl.ds(p1 + k * LANES, LANES)], sem)
            for k in range(CHUNK_ROWS):
                pltpu.sync_copy(rows_v.at[pl.ds(p0 + k * LANES, LANES)],
                                acc.at[dst_v.at[b0 * CHUNK_ROWS + k]],
                                add=True)
        plsc.subcore_barrier()

        pltpu.sync_copy(
            acc.at[pl.ds(sub * ROWS_PER_SUB, ROWS_PER_SUB)],
            tmp.at[pl.ds(rel * NP_PAD + sub * ROWS_PER_SUB, ROWS_PER_SUB),
                   pl.ds(g * CW, CW)])

    for rel in range(3):
        for g in range(NG):
            @pl.when(core == (g // (NG // NC)))
            def _(rel=rel, g=g):
                round_body(tables[rel * NG + g], srcs[rel], dsts[rel],
                           rel, g)


def _sc_aggregate(tables, srcs, dsts):
    mesh = plsc.VectorSubcoreMesh(core_axis_name="c", subcore_axis_name="s")
    kern = pl.kernel(
        _sc_kernel,
        out_type=jax.ShapeDtypeStruct((3 * NP_PAD, D), jnp.float32),
        mesh=mesh,
        compiler_params=pltpu.CompilerParams(use_tc_tiling_on_sc=False),
        scratch_types=[
            pltpu.VMEM((3 * CHUNK_ROWS, LANES), jnp.int32),
            pltpu.VMEM((3 * CHUNK_ROWS, LANES), jnp.int32),
            pltpu.VMEM((2 * CHUNK, CW), jnp.float32),
            pltpu.VMEM((ZB_ROWS, CW), jnp.float32),
            pltpu.VMEM_SHARED((N_GO, CW), jnp.float32),
            pltpu.VMEM_SHARED((ACC_ROWS, CW), jnp.float32),
            pltpu.SemaphoreType.DMA,
            pltpu.SemaphoreType.DMA,
        ],
    )
    return kern(*tables, *srcs, *dsts)


def _combine_kernel(mf_ref, bp_ref, cc_ref, b_ref, o_ref):
    o_ref[...] = (
        jnp.maximum(mf_ref[0] + b_ref[0], 0.0)
        + jnp.maximum(bp_ref[0] + b_ref[1], 0.0)
        + jnp.maximum(cc_ref[0] + b_ref[2], 0.0))


def _combine(tmp, b3):
    tmp = tmp.reshape(3, NP_PAD, D)
    grid = (N_P // MB,)
    in_specs = [
        pl.BlockSpec((1, MB, D), lambda i: (0, i, 0)),
        pl.BlockSpec((1, MB, D), lambda i: (1, i, 0)),
        pl.BlockSpec((1, MB, D), lambda i: (2, i, 0)),
        pl.BlockSpec((3, D), lambda i: (0, 0)),
    ]
    return pl.pallas_call(
        _combine_kernel,
        grid=grid,
        in_specs=in_specs,
        out_specs=pl.BlockSpec((MB, D), lambda i: (i, 0)),
        out_shape=jax.ShapeDtypeStruct((N_P, D), jnp.float32),
    )(tmp, tmp, tmp, b3)


def _pad_idx(idx, fill):
    idx = idx.astype(jnp.int32)
    return jnp.concatenate([idx, fill]).reshape(IDX_ROWS, LANES)


@jax.jit
def _impl(h_mf, h_bp, h_cc, src_mf, dst_mf, src_bp, dst_bp, src_cc, dst_cc,
          W_mf, b_mf, W_bp, b_bp, W_cc, b_cc):
    tables = []
    for h, W in ((h_mf, W_mf), (h_bp, W_bp), (h_cc, W_cc)):
        tables.extend(_feat_tables(h, W))
    ar = jnp.arange(E_PAD - E, dtype=jnp.int32)
    src_fill = ar % N_GO
    dst_fill = NP_PAD + (ar % (ACC_ROWS - NP_PAD))
    srcs = [_pad_idx(s, src_fill) for s in (src_mf, src_bp, src_cc)]
    dsts = [_pad_idx(d, dst_fill) for d in (dst_mf, dst_bp, dst_cc)]
    tmp = _sc_aggregate(tables, srcs, dsts)
    b3 = jnp.stack([b_mf, b_bp, b_cc])
    return _combine(tmp, b3)


def kernel(h_p, h_mf, h_bp, h_cc, src_mf, dst_mf, src_bp, dst_bp,
           src_cc, dst_cc, W_mf, b_mf, W_bp, b_bp, W_cc, b_cc):
    del h_p
    return _impl(h_mf, h_bp, h_cc, src_mf, dst_mf, src_bp, dst_bp,
                 src_cc, dst_cc, W_mf, b_mf, W_bp, b_bp, W_cc, b_cc)


# speedup vs baseline: 6.0944x; 1.2009x over previous
import jax
import jax.numpy as jnp
from jax import lax
from jax.experimental import pallas as pl
from jax.experimental.pallas import tpu as pltpu
from jax.experimental.pallas import tpu_sc as plsc

N_P = 50000
N_GO = 10000
E = 500000
D = 128

NG = 8
CW = D // NG
NC = 2
NS = 16
LANES = 128
CHUNK_ROWS = 4
CHUNK = CHUNK_ROWS * LANES
CHUNKS_PER_SUB = 62
E_PAD = NS * CHUNKS_PER_SUB * CHUNK
IDX_ROWS = E_PAD // LANES
ROWS_PER_SUB = 3128
NP_PAD = NS * ROWS_PER_SUB
ACC_ROWS = NP_PAD + 16
ZB_ROWS = 136
N_ZERO_COPIES = ROWS_PER_SUB // ZB_ROWS
TAB_SPAN = 632
TAB_TAIL = N_GO - 15 * TAB_SPAN

MB = 2000


def _matmul_kernel(h_ref, w_ref, *outs):
    f = lax.dot_general(h_ref[...], w_ref[...], (((1,), (0,)), ((), ())),
                        precision=lax.Precision.HIGHEST,
                        preferred_element_type=jnp.float32)
    for g in range(NG):
        outs[g][...] = f[:, g * CW:(g + 1) * CW]


def _feat_tables(h, W):
    grid = (N_GO // MB,)
    out_shapes = tuple(jax.ShapeDtypeStruct((N_GO, CW), jnp.float32)
                       for _ in range(NG))
    return pl.pallas_call(
        _matmul_kernel,
        grid=grid,
        in_specs=[
            pl.BlockSpec((MB, D), lambda i: (i, 0)),
            pl.BlockSpec((D, D), lambda i: (0, 0)),
        ],
        out_specs=tuple(pl.BlockSpec((MB, CW), lambda i: (i, 0))
                        for _ in range(NG)),
        out_shape=out_shapes,
    )(h, W)


def _sc_kernel(*refs):
    nt = 3 * NG
    tables = refs[0:nt]
    srcs = refs[nt:nt + 3]
    dsts = refs[nt + 3:nt + 6]
    tmp = refs[nt + 6]
    src_v, dst_v, rows_v, zb, tab_s, acc, sem, isem, ssem = (
        refs[nt + 7:nt + 16])

    core = lax.axis_index("c")
    sub = lax.axis_index("s")

    zrow = jnp.zeros((16,), jnp.float32)
    @pl.loop(0, ZB_ROWS)
    def _(r):
        zb[r, 0:CW] = zrow

    def round_body(tab, src_hbm, dst_hbm, rel, g):
        @pl.when(sub < 15)
        def _():
            pltpu.sync_copy(tab.at[pl.ds(sub * TAB_SPAN, TAB_SPAN)],
                            tab_s.at[pl.ds(sub * TAB_SPAN, TAB_SPAN)])
        @pl.when(sub == 15)
        def _():
            pltpu.sync_copy(tab.at[pl.ds(15 * TAB_SPAN, TAB_TAIL)],
                            tab_s.at[pl.ds(15 * TAB_SPAN, TAB_TAIL)])
        @pl.loop(0, N_ZERO_COPIES)
        def _(i):
            pltpu.sync_copy(
                zb.at[pl.ds(0, ZB_ROWS)],
                acc.at[pl.ds(sub * ROWS_PER_SUB + i * ZB_ROWS, ZB_ROWS)])
        @pl.when(sub == 0)
        def _():
            pltpu.sync_copy(zb.at[pl.ds(0, ACC_ROWS - NP_PAD)],
                            acc.at[pl.ds(NP_PAD, ACC_ROWS - NP_PAD)])
        plsc.subcore_barrier()

        base = sub * (CHUNKS_PER_SUB * CHUNK_ROWS)

        def fire_idx(chunk, buf, how):
            row0 = base + chunk * CHUNK_ROWS
            how(src_hbm.at[pl.ds(row0, CHUNK_ROWS)],
                src_v.at[pl.ds(buf * CHUNK_ROWS, CHUNK_ROWS)])
            how(dst_hbm.at[pl.ds(row0, CHUNK_ROWS)],
                dst_v.at[pl.ds(buf * CHUNK_ROWS, CHUNK_ROWS)])

        fire_idx(0, 0, pltpu.sync_copy)
        fire_idx(1, 1, pltpu.sync_copy)
        for k in range(CHUNK_ROWS):
            pltpu.async_copy(tab_s.at[src_v.at[k]],
                             rows_v.at[pl.ds(k * LANES, LANES)], sem)

        @pl.loop(0, CHUNKS_PER_SUB)
        def _(j):
            i0 = lax.rem(j, 4)
            i2 = lax.rem(j + 2, 4)
            r0 = lax.rem(j, 3)
            r1 = lax.rem(j + 1, 3)
            @pl.when(j >= 2)
            def _():
                for k in range(CHUNK_ROWS):
                    pltpu.make_async_copy(
                        rows_v.at[pl.ds(k * LANES, LANES)],
                        acc.at[dst_v.at[k]], ssem).wait()
            @pl.when(jnp.logical_and(j >= 1, j + 1 < CHUNKS_PER_SUB))
            def _():
                for _i in range(2):
                    pltpu.make_async_copy(
                        src_hbm.at[pl.ds(base, CHUNK_ROWS)],
                        src_v.at[pl.ds(0, CHUNK_ROWS)], isem).wait()
            @pl.when(j + 2 < CHUNKS_PER_SUB)
            def _():
                fire_idx(j + 2, i2,
                         lambda s, d: pltpu.async_copy(s, d, isem))
            for k in range(CHUNK_ROWS):
                pltpu.make_async_copy(
                    tab_s.at[src_v.at[i0 * CHUNK_ROWS + k]],
                    rows_v.at[pl.ds(r0 * CHUNK + k * LANES, LANES)],
                    sem).wait()
            @pl.when(j + 1 < CHUNKS_PER_SUB)
            def _():
                for k in range(CHUNK_ROWS):
                    pltpu.async_copy(
                        tab_s.at[src_v.at[lax.rem(j + 1, 4) * CHUNK_ROWS + k]],
                        rows_v.at[pl.ds(r1 * CHUNK + k * LANES, LANES)], sem)
            for k in range(CHUNK_ROWS):
                pltpu.async_copy(
                    rows_v.at[pl.ds(r0 * CHUNK + k * LANES, LANES)],
                    acc.at[dst_v.at[i0 * CHUNK_ROWS + k]], ssem, add=True)
        for _i in range(2 * CHUNK_ROWS):
            pltpu.make_async_copy(rows_v.at[pl.ds(0, LANES)],
                                  acc.at[dst_v.at[0]], ssem).wait()
        plsc.subcore_barrier()

        pltpu.sync_copy(
            acc.at[pl.ds(sub * ROWS_PER_SUB, ROWS_PER_SUB)],
            tmp.at[pl.ds(rel * NP_PAD + sub * ROWS_PER_SUB, ROWS_PER_SUB),
                   pl.ds(g * CW, CW)])

    for rel in range(3):
        for g in range(NG):
            @pl.when(core == (g // (NG // NC)))
            def _(rel=rel, g=g):
                round_body(tables[rel * NG + g], srcs[rel], dsts[rel],
                           rel, g)


def _sc_aggregate(tables, srcs, dsts):
    mesh = plsc.VectorSubcoreMesh(core_axis_name="c", subcore_axis_name="s")
    kern = pl.kernel(
        _sc_kernel,
        out_type=jax.ShapeDtypeStruct((3 * NP_PAD, D), jnp.float32),
        mesh=mesh,
        compiler_params=pltpu.CompilerParams(use_tc_tiling_on_sc=False),
        scratch_types=[
            pltpu.VMEM((4 * CHUNK_ROWS, LANES), jnp.int32),
            pltpu.VMEM((4 * CHUNK_ROWS, LANES), jnp.int32),
            pltpu.VMEM((3 * CHUNK, CW), jnp.float32),
            pltpu.VMEM((ZB_ROWS, CW), jnp.float32),
            pltpu.VMEM_SHARED((N_GO, CW), jnp.float32),
            pltpu.VMEM_SHARED((ACC_ROWS, CW), jnp.float32),
            pltpu.SemaphoreType.DMA,
            pltpu.SemaphoreType.DMA,
            pltpu.SemaphoreType.DMA,
        ],
    )
    return kern(*tables, *srcs, *dsts)


def _combine_kernel(mf_ref, bp_ref, cc_ref, b_ref, o_ref):
    o_ref[...] = (
        jnp.maximum(mf_ref[0] + b_ref[0], 0.0)
        + jnp.maximum(bp_ref[0] + b_ref[1], 0.0)
        + jnp.maximum(cc_ref[0] + b_ref[2], 0.0))


def _combine(tmp, b3):
    tmp = tmp.reshape(3, NP_PAD, D)
    grid = (N_P // MB,)
    in_specs = [
        pl.BlockSpec((1, MB, D), lambda i: (0, i, 0)),
        pl.BlockSpec((1, MB, D), lambda i: (1, i, 0)),
        pl.BlockSpec((1, MB, D), lambda i: (2, i, 0)),
        pl.BlockSpec((3, D), lambda i: (0, 0)),
    ]
    return pl.pallas_call(
        _combine_kernel,
        grid=grid,
        in_specs=in_specs,
        out_specs=pl.BlockSpec((MB, D), lambda i: (i, 0)),
        out_shape=jax.ShapeDtypeStruct((N_P, D), jnp.float32),
    )(tmp, tmp, tmp, b3)


def _pad_idx(idx, fill):
    idx = idx.astype(jnp.int32)
    return jnp.concatenate([idx, fill]).reshape(IDX_ROWS, LANES)


@jax.jit
def _impl(h_mf, h_bp, h_cc, src_mf, dst_mf, src_bp, dst_bp, src_cc, dst_cc,
          W_mf, b_mf, W_bp, b_bp, W_cc, b_cc):
    tables = []
    for h, W in ((h_mf, W_mf), (h_bp, W_bp), (h_cc, W_cc)):
        tables.extend(_feat_tables(h, W))
    ar = jnp.arange(E_PAD - E, dtype=jnp.int32)
    src_fill = ar % N_GO
    dst_fill = NP_PAD + (ar % (ACC_ROWS - NP_PAD))
    srcs = [_pad_idx(s, src_fill) for s in (src_mf, src_bp, src_cc)]
    dsts = [_pad_idx(d, dst_fill) for d in (dst_mf, dst_bp, dst_cc)]
    tmp = _sc_aggregate(tables, srcs, dsts)
    b3 = jnp.stack([b_mf, b_bp, b_cc])
    return _combine(tmp, b3)


def kernel(h_p, h_mf, h_bp, h_cc, src_mf, dst_mf, src_bp, dst_bp,
           src_cc, dst_cc, W_mf, b_mf, W_bp, b_bp, W_cc, b_cc):
    del h_p
    return _impl(h_mf, h_bp, h_cc, src_mf, dst_mf, src_bp, dst_bp,
                 src_cc, dst_cc, W_mf, b_mf, W_bp, b_bp, W_cc, b_cc)
